# Initial kernel scaffold; baseline (speedup 1.0000x reference)
import jax
import jax.numpy as jnp
from jax.experimental import pallas as pl


def kernel(x, edge_index, edge_attr, batch, W0, b0, Wc0, bc0, Wc1, bc1, Wc2, bc2, g0, bt0, g1, bt1, g2, bt2, W1, b1, W2, b2, W3, b3):
    raise NotImplementedError("write your pallas kernel here")


# speedup vs baseline: 12.9853x; 12.9853x over previous
import functools

import jax
import jax.numpy as jnp
from jax import lax
from jax.experimental import pallas as pl
from jax.experimental.pallas import tpu as pltpu
from jax.experimental.pallas import tpu_sc as plsc

_N = 10000
_E = 320000
_H = 128
_G = 64
_EPS = 1e-5

_C = 100
_NCH = _E // _C
_NW = 32
_CPW = _NCH // _NW
_IB = 20
_NB = _CPW // _IB
_NP = 10240
_RPS = _NP // 16

_MESH = plsc.VectorSubcoreMesh(core_axis_name="c", subcore_axis_name="s")


def _leaky(v):
    return jnp.where(v >= 0, v, 0.01 * v)


def _sc_degree(dst2d):
    @functools.partial(
        pl.kernel,
        out_type=jax.ShapeDtypeStruct((2, _NP, _H), jnp.float32),
        mesh=_MESH,
        scratch_types=[
            pltpu.VMEM((_IB, _C), jnp.int32),
            pltpu.VMEM((_C, _H), jnp.float32),
            pltpu.VMEM_SHARED((_NP, _H), jnp.float32),
        ],
    )
    def k(dst_hbm, out_hbm, dstv, onesv, acc):
        c = lax.axis_index("c")
        s = lax.axis_index("s")
        row0 = s * _RPS

        def fill(i, val):
            for k2 in range(_H // 16):
                onesv[i, pl.ds(k2 * 16, 16)] = jnp.full((16,), val, jnp.float32)
            return val

        lax.fori_loop(0, _C, fill, 0.0)

        def zc(t, _):
            pltpu.sync_copy(onesv, acc.at[pl.ds(row0 + t * _C, _C)])
            return 0

        lax.fori_loop(0, _RPS // _C, zc, 0)
        _tail = _RPS - (_RPS // _C) * _C
        pltpu.sync_copy(onesv.at[pl.ds(0, _tail)],
                        acc.at[pl.ds(row0 + (_RPS // _C) * _C, _tail)])

        lax.fori_loop(0, _C, fill, 1.0)

        wid = c * 16 + s
        plsc.subcore_barrier()

        def blk(b, _):
            pltpu.sync_copy(dst_hbm.at[wid, b], dstv)

            def body(j, _2):
                pltpu.sync_copy(onesv, acc.at[dstv.at[j]], add=True)
                return 0

            lax.fori_loop(0, _IB, body, 0)
            return 0

        lax.fori_loop(0, _NB, blk, 0)
        plsc.subcore_barrier()
        pltpu.sync_copy(acc.at[pl.ds(row0, _RPS)],
                        out_hbm.at[c, pl.ds(row0, _RPS)])

    return k(dst2d)


def _sc_msg(y, src2d, dst2d):
    @functools.partial(
        pl.kernel,
        out_type=jax.ShapeDtypeStruct((2, _NP, _H), jnp.float32),
        mesh=_MESH,
        scratch_types=[
            pltpu.VMEM((_IB, _C), jnp.int32),
            pltpu.VMEM((_IB, _C), jnp.int32),
            pltpu.VMEM((_C, _H), jnp.float32),
            pltpu.VMEM((_C, _H), jnp.float32),
            pltpu.VMEM_SHARED((_NP, _H), jnp.float32),
            pltpu.SemaphoreType.DMA,
        ],
    )
    def k(y_hbm, src_hbm, dst_hbm, out_hbm, srcv, dstv, rowsA, rowsB, acc, gsem):
        c = lax.axis_index("c")
        s = lax.axis_index("s")
        row0 = s * _RPS

        def z1(i, _):
            for k2 in range(_H // 16):
                rowsA[i, pl.ds(k2 * 16, 16)] = jnp.zeros((16,), jnp.float32)
            return 0

        lax.fori_loop(0, _C, z1, 0)

        def zc(t, _):
            pltpu.sync_copy(rowsA, acc.at[pl.ds(row0 + t * _C, _C)])
            return 0

        lax.fori_loop(0, _RPS // _C, zc, 0)
        _tail = _RPS - (_RPS // _C) * _C
        pltpu.sync_copy(rowsA.at[pl.ds(0, _tail)],
                        acc.at[pl.ds(row0 + (_RPS // _C) * _C, _tail)])

        wid = c * 16 + s
        plsc.subcore_barrier()

        def blk(b, _):
            pltpu.sync_copy(src_hbm.at[wid, b], srcv)
            pltpu.sync_copy(dst_hbm.at[wid, b], dstv)

            def body(jj, _2):
                j = jj * 2
                pltpu.async_copy(y_hbm.at[srcv.at[j]], rowsA, gsem).wait()
                cpB = pltpu.async_copy(y_hbm.at[srcv.at[j + 1]], rowsB, gsem)
                pltpu.sync_copy(rowsA, acc.at[dstv.at[j]], add=True)
                cpB.wait()
                pltpu.sync_copy(rowsB, acc.at[dstv.at[j + 1]], add=True)
                return 0

            lax.fori_loop(0, _IB // 2, body, 0)
            return 0

        lax.fori_loop(0, _NB, blk, 0)
        plsc.subcore_barrier()
        pltpu.sync_copy(acc.at[pl.ds(row0, _RPS)],
                        out_hbm.at[c, pl.ds(row0, _RPS)])

    return k(y, src2d, dst2d)


def _mask_of(batch_row):
    iota = lax.broadcasted_iota(jnp.int32, (_G, _N), 0)
    return (iota == batch_row).astype(jnp.float32)


def _dotT(a, b):
    return lax.dot_general(a, b, (((0,), (0,)), ((), ())),
                           preferred_element_type=jnp.float32)


def _dot(a, b):
    return jnp.dot(a, b, preferred_element_type=jnp.float32)


def _tc_init_body(x_ref, w0_ref, b0_ref, batch_ref, deg_ref,
                  h_ref, gap_ref, dis_ref, counts_ref):
    mask = _mask_of(batch_ref[...])
    h = _leaky(_dot(x_ref[...], w0_ref[...]) + b0_ref[...])
    h_ref[...] = h
    counts = jnp.sum(mask, axis=1, keepdims=True)
    counts_ref[...] = counts
    gap_ref[...] = _dot(mask, h) / jnp.maximum(counts, 1.0)
    deg = deg_ref[0, 0:_N, 0:1] + deg_ref[1, 0:_N, 0:1] + 1.0
    dis_ref[...] = lax.rsqrt(deg)


def _tc_init(x, W0, b0r, batch_row, deg_parts):
    return pl.pallas_call(
        _tc_init_body,
        out_shape=[
            jax.ShapeDtypeStruct((_N, _H), jnp.float32),
            jax.ShapeDtypeStruct((_G, _H), jnp.float32),
            jax.ShapeDtypeStruct((_N, 1), jnp.float32),
            jax.ShapeDtypeStruct((_G, 1), jnp.float32),
        ],
    )(x, W0, b0r, batch_row, deg_parts)


def _tc_a_body(h_ref, gap_ref, dis_ref, batch_ref, wt_ref, wb_ref, y_ref):
    mask = _mask_of(batch_ref[...])
    gw = _dot(gap_ref[...], wb_ref[...])
    xw = _dot(h_ref[...], wt_ref[...]) + _dotT(mask, gw)
    y_ref[...] = xw * dis_ref[...]


def _tc_a(h, gap, dis, batch_row, Wt, Wb):
    return pl.pallas_call(
        _tc_a_body,
        out_shape=jax.ShapeDtypeStruct((_N, _H), jnp.float32),
    )(h, gap, dis, batch_row, Wt, Wb)


def _tc_b_body(msum_ref, y_ref, dis_ref, batch_ref, batchT_ref, bc_ref,
               g_ref, bt_ref, counts_ref, pooled_in_ref,
               h_ref, gap_ref, pooled_ref, gmp_ref):
    mask = _mask_of(batch_ref[...])
    counts = counts_ref[...]
    dis = dis_ref[...]
    t = dis * (msum_ref[0, 0:_N] + msum_ref[1, 0:_N] + y_ref[...]) + bc_ref[...]
    denom = jnp.maximum(counts, 1.0) * float(_H)
    m1 = jnp.sum(_dot(mask, t), axis=1, keepdims=True) / denom
    m1n = _dotT(mask, m1)
    xc = t - m1n
    var = jnp.sum(_dot(mask, xc * xc), axis=1, keepdims=True) / denom
    rv = lax.rsqrt(var + _EPS)
    rvn = _dotT(mask, rv)
    h = _leaky(xc * rvn * g_ref[...] + bt_ref[...])
    h_ref[...] = h
    gap = _dot(mask, h) / jnp.maximum(counts, 1.0)
    gap_ref[...] = gap

    bT = batchT_ref[...]

    def gbody(g, _):
        mg = jnp.max(jnp.where(bT == g, h, -1e30), axis=0, keepdims=True)
        cg = counts_ref[pl.ds(g, 1), :]
        gmp_ref[pl.ds(g, 1), :] = jnp.where(cg > 0, mg, 0.0)
        return 0

    lax.fori_loop(0, _G, gbody, 0)
    pooled_ref[...] = pooled_in_ref[...] + jnp.concatenate(
        [gmp_ref[...], gap], axis=1)


def _tc_b(msum, y, dis, batch_row, batchT, bcr, gr, btr, counts, pooled_in):
    h, gap, pooled, _ = pl.pallas_call(
        _tc_b_body,
        out_shape=[
            jax.ShapeDtypeStruct((_N, _H), jnp.float32),
            jax.ShapeDtypeStruct((_G, _H), jnp.float32),
            jax.ShapeDtypeStruct((_G, 2 * _H), jnp.float32),
            jax.ShapeDtypeStruct((_G, _H), jnp.float32),
        ],
    )(msum, y, dis, batch_row, batchT, bcr, gr, btr, counts, pooled_in)
    return h, gap, pooled


def _tc_final_body(p_ref, w1_ref, b1_ref, w2_ref, b2_ref, w3_ref, b3_ref,
                   out_ref):
    o = _leaky(_dot(p_ref[...], w1_ref[...]) + b1_ref[...])
    o = _leaky(_dot(o, w2_ref[...]) + b2_ref[...])
    out_ref[...] = _dot(o, w3_ref[...]) + b3_ref[...]


def _tc_final(pooled, W1, b1r, W2, b2r, W3, b3r):
    return pl.pallas_call(
        _tc_final_body,
        out_shape=jax.ShapeDtypeStruct((_G, 1), jnp.float32),
    )(pooled, W1, b1r, W2, b2r, W3, b3r)


def kernel(x, edge_index, edge_attr, batch, W0, b0, Wc0, bc0, Wc1, bc1,
           Wc2, bc2, g0, bt0, g1, bt1, g2, bt2, W1, b1, W2, b2, W3, b3):
    src2d = edge_index[0].reshape(_NW, _NB, _IB, _C)
    dst2d = edge_index[1].reshape(_NW, _NB, _IB, _C)
    batch_row = batch.reshape(1, _N)
    batchT = batch.reshape(_N, 1)

    deg_parts = _sc_degree(dst2d)
    h, gap, dis, counts = _tc_init(x, W0, b0.reshape(1, _H), batch_row,
                                   deg_parts)

    pooled = jnp.zeros((_G, 2 * _H), jnp.float32)
    layer_params = [(Wc0, bc0, g0, bt0), (Wc1, bc1, g1, bt1),
                    (Wc2, bc2, g2, bt2)]
    for Wc, bc, g, bt in layer_params:
        y = _tc_a(h, gap, dis, batch_row, Wc[:_H], Wc[_H:])
        msum = _sc_msg(y, src2d, dst2d)
        h, gap, pooled = _tc_b(msum, y, dis, batch_row, batchT,
                               bc.reshape(1, _H), g.reshape(1, _H),
                               bt.reshape(1, _H), counts, pooled)

    return _tc_final(pooled, W1, b1.reshape(1, 4 * _H),
                     W2, b2.reshape(1, 4 * _H), W3, b3.reshape(1, 1))


# speedup vs baseline: 14.1471x; 1.0895x over previous
import functools

import jax
import jax.numpy as jnp
from jax import lax
from jax.experimental import pallas as pl
from jax.experimental.pallas import tpu as pltpu
from jax.experimental.pallas import tpu_sc as plsc

_N = 10000
_E = 320000
_H = 128
_G = 64
_EPS = 1e-5

_NW = 32
_EPW = _E // _NW
_NP = 10240
_RPS = _NP // 16

_C = 50
_MIB = 8
_MNB = _EPW // (_MIB * _C)

_DC = 100
_DIB = 10
_DNB = _EPW // (_DIB * _DC)

_MESH = plsc.VectorSubcoreMesh(core_axis_name="c", subcore_axis_name="s")


def _leaky(v):
    return jnp.where(v >= 0, v, 0.01 * v)


def _sc_degree(dst2d):
    @functools.partial(
        pl.kernel,
        out_type=jax.ShapeDtypeStruct((2, _NP, _H), jnp.float32),
        mesh=_MESH,
        scratch_types=[
            pltpu.VMEM((_DIB, _DC), jnp.int32),
            pltpu.VMEM((_DC, _H), jnp.float32),
            pltpu.VMEM_SHARED((_NP, _H), jnp.float32),
            pltpu.SemaphoreType.DMA,
        ],
    )
    def k(dst_hbm, out_hbm, dstv, onesv, acc, ssem):
        c = lax.axis_index("c")
        s = lax.axis_index("s")
        row0 = s * _RPS

        def fill(i, val):
            for k2 in range(_H // 16):
                onesv[i, pl.ds(k2 * 16, 16)] = jnp.full((16,), val, jnp.float32)
            return val

        lax.fori_loop(0, _DC, fill, 0.0)

        def zc(t, _):
            pltpu.sync_copy(onesv, acc.at[pl.ds(row0 + t * _DC, _DC)])
            return 0

        lax.fori_loop(0, _RPS // _DC, zc, 0)
        _tail = _RPS - (_RPS // _DC) * _DC
        pltpu.sync_copy(onesv.at[pl.ds(0, _tail)],
                        acc.at[pl.ds(row0 + (_RPS // _DC) * _DC, _tail)])

        lax.fori_loop(0, _DC, fill, 1.0)

        wid = c * 16 + s
        plsc.subcore_barrier()

        def blk(b, _):
            pltpu.sync_copy(dst_hbm.at[wid, b], dstv)
            for j in range(_DIB):
                pltpu.async_copy(onesv, acc.at[dstv.at[j]], ssem, add=True)
            for j in range(_DIB):
                pltpu.make_async_copy(onesv, acc.at[pl.ds(row0, _DC)],
                                      ssem).wait()
            return 0

        lax.fori_loop(0, _DNB, blk, 0)
        plsc.subcore_barrier()
        pltpu.sync_copy(acc.at[pl.ds(row0, _RPS)],
                        out_hbm.at[c, pl.ds(row0, _RPS)])

    return k(dst2d)


def _sc_msg(y, src4, dst4):
    @functools.partial(
        pl.kernel,
        out_type=jax.ShapeDtypeStruct((2, _NP, _H), jnp.float32),
        mesh=_MESH,
        scratch_types=[
            pltpu.VMEM((_MIB, _C), jnp.int32),
            pltpu.VMEM((_MIB, _C), jnp.int32),
            pltpu.VMEM((_MIB, _C), jnp.int32),
            pltpu.VMEM((_MIB, _C), jnp.int32),
            pltpu.VMEM((_MIB, _C), jnp.int32),
            pltpu.VMEM((_MIB, _C), jnp.int32),
            pltpu.VMEM((_C, _H), jnp.float32),
            pltpu.VMEM((_C, _H), jnp.float32),
            pltpu.VMEM((_C, _H), jnp.float32),
            pltpu.VMEM((_C, _H), jnp.float32),
            pltpu.VMEM_SHARED((_NP, _H), jnp.float32),
            pltpu.SemaphoreType.DMA,
            pltpu.SemaphoreType.DMA,
            pltpu.SemaphoreType.DMA,
        ],
    )
    def k(y_hbm, src_hbm, dst_hbm, out_hbm, sI0, sI1, sI2, dI0, dI1, dI2,
          R0, R1, R2, R3, acc, gsem, ssem, isem):
        c = lax.axis_index("c")
        s = lax.axis_index("s")
        row0 = s * _RPS
        wid = c * 16 + s
        sI = (sI0, sI1, sI2)
        dI = (dI0, dI1, dI2)
        R = (R0, R1, R2, R3)

        def ig(ib, j, r):
            pltpu.async_copy(y_hbm.at[sI[ib].at[j]], R[r], gsem)

        def isc(ib, j, r):
            pltpu.async_copy(R[r], acc.at[dI[ib].at[j]], ssem, add=True)

        def wg(r):
            pltpu.make_async_copy(acc.at[pl.ds(row0, _C)], R[r], gsem).wait()

        def ws():
            pltpu.make_async_copy(R0, acc.at[pl.ds(row0, _C)], ssem).wait()

        def wi():
            pltpu.make_async_copy(src_hbm.at[wid, 0], sI0, isem).wait()
            pltpu.make_async_copy(dst_hbm.at[wid, 0], dI0, isem).wait()

        def pfi(b, ib):
            pltpu.async_copy(src_hbm.at[wid, b], sI[ib], isem)
            pltpu.async_copy(dst_hbm.at[wid, b], dI[ib], isem)

        pltpu.sync_copy(src_hbm.at[wid, 0], sI0)
        pltpu.sync_copy(dst_hbm.at[wid, 0], dI0)
        pfi(1, 1)
        pfi(2, 2)

        def z1(i, _):
            for k2 in range(_H // 16):
                R0[i, pl.ds(k2 * 16, 16)] = jnp.zeros((16,), jnp.float32)
            return 0

        lax.fori_loop(0, _C, z1, 0)

        def zc(t, _):
            pltpu.sync_copy(R0, acc.at[pl.ds(row0 + t * _C, _C)])
            return 0

        lax.fori_loop(0, _RPS // _C, zc, 0)
        _tail = _RPS - (_RPS // _C) * _C
        pltpu.sync_copy(R0.at[pl.ds(0, _tail)],
                        acc.at[pl.ds(row0 + (_RPS // _C) * _C, _tail)])
        plsc.subcore_barrier()

        ig(0, 0, 0); ig(0, 1, 1); ig(0, 2, 2); ig(0, 3, 3)
        wg(0); wg(1)
        isc(0, 0, 0); isc(0, 1, 1)
        wg(2); wg(3)
        ws(); ws()
        ig(0, 4, 0); ig(0, 5, 1)
        isc(0, 2, 2); isc(0, 3, 3)

        def body(m, _):
            for d in range(3):
                b = 3 * m + d
                ibc = d
                ibn = (d + 1) % 3
                wi()
                for u in range(4):
                    cur = (ibc, 4 + 2 * u) if u < 2 else (ibn, 2 * u - 4)
                    nxt = (ibc, 6) if u == 0 else (ibn, 2 * u - 2)
                    r0, r1 = (0, 1) if u % 2 == 0 else (2, 3)
                    z0, z1_ = (2, 3) if u % 2 == 0 else (0, 1)
                    wg(r0); wg(r1)
                    ws(); ws()
                    ig(nxt[0], nxt[1], z0); ig(nxt[0], nxt[1] + 1, z1_)
                    isc(cur[0], cur[1], r0); isc(cur[0], cur[1] + 1, r1)

                @pl.when(b < 22)
                def _pf():
                    pfi(b + 3, ibc)

            return 0

        lax.fori_loop(0, 8, body, 0)

        wg(0); wg(1)
        ws(); ws()
        ig(0, 6, 2); ig(0, 7, 3)
        isc(0, 4, 0); isc(0, 5, 1)
        wg(2); wg(3)
        ws(); ws()
        isc(0, 6, 2); isc(0, 7, 3)
        ws(); ws()

        plsc.subcore_barrier()
        pltpu.sync_copy(acc.at[pl.ds(row0, _RPS)],
                        out_hbm.at[c, pl.ds(row0, _RPS)])

    return k(y, src4, dst4)


def _mask_of(batch_row):
    iota = lax.broadcasted_iota(jnp.int32, (_G, _N), 0)
    return (iota == batch_row).astype(jnp.float32)


def _dotT(a, b):
    return lax.dot_general(a, b, (((0,), (0,)), ((), ())),
                           preferred_element_type=jnp.float32)


def _dot(a, b):
    return jnp.dot(a, b, preferred_element_type=jnp.float32)


def _tc_init_body(x_ref, w0_ref, b0_ref, batch_ref, deg_ref,
                  h_ref, gap_ref, dis_ref, counts_ref):
    mask = _mask_of(batch_ref[...])
    h = _leaky(_dot(x_ref[...], w0_ref[...]) + b0_ref[...])
    h_ref[...] = h
    counts = jnp.sum(mask, axis=1, keepdims=True)
    counts_ref[...] = counts
    gap_ref[...] = _dot(mask, h) / jnp.maximum(counts, 1.0)
    deg = deg_ref[0, 0:_N, 0:1] + deg_ref[1, 0:_N, 0:1] + 1.0
    dis_ref[...] = lax.rsqrt(deg)


def _tc_init(x, W0, b0r, batch_row, deg_parts):
    return pl.pallas_call(
        _tc_init_body,
        out_shape=[
            jax.ShapeDtypeStruct((_N, _H), jnp.float32),
            jax.ShapeDtypeStruct((_G, _H), jnp.float32),
            jax.ShapeDtypeStruct((_N, 1), jnp.float32),
            jax.ShapeDtypeStruct((_G, 1), jnp.float32),
        ],
    )(x, W0, b0r, batch_row, deg_parts)


def _tc_a_body(h_ref, gap_ref, dis_ref, batch_ref, wt_ref, wb_ref, y_ref):
    mask = _mask_of(batch_ref[...])
    gw = _dot(gap_ref[...], wb_ref[...])
    xw = _dot(h_ref[...], wt_ref[...]) + _dotT(mask, gw)
    y_ref[...] = xw * dis_ref[...]


def _tc_a(h, gap, dis, batch_row, Wt, Wb):
    return pl.pallas_call(
        _tc_a_body,
        out_shape=jax.ShapeDtypeStruct((_N, _H), jnp.float32),
    )(h, gap, dis, batch_row, Wt, Wb)


def _tc_b_body(msum_ref, y_ref, dis_ref, batch_ref, batchT_ref, bc_ref,
               g_ref, bt_ref, counts_ref, pooled_in_ref,
               h_ref, gap_ref, pooled_ref, gmp_ref):
    mask = _mask_of(batch_ref[...])
    counts = counts_ref[...]
    dis = dis_ref[...]
    t = dis * (msum_ref[0, 0:_N] + msum_ref[1, 0:_N] + y_ref[...]) + bc_ref[...]
    denom = jnp.maximum(counts, 1.0) * float(_H)
    m1 = jnp.sum(_dot(mask, t), axis=1, keepdims=True) / denom
    m1n = _dotT(mask, m1)
    xc = t - m1n
    var = jnp.sum(_dot(mask, xc * xc), axis=1, keepdims=True) / denom
    rv = lax.rsqrt(var + _EPS)
    rvn = _dotT(mask, rv)
    h = _leaky(xc * rvn * g_ref[...] + bt_ref[...])
    h_ref[...] = h
    gap = _dot(mask, h) / jnp.maximum(counts, 1.0)
    gap_ref[...] = gap

    bT = batchT_ref[...]

    def gbody(g, _):
        mg = jnp.max(jnp.where(bT == g, h, -1e30), axis=0, keepdims=True)
        cg = counts_ref[pl.ds(g, 1), :]
        gmp_ref[pl.ds(g, 1), :] = jnp.where(cg > 0, mg, 0.0)
        return 0

    lax.fori_loop(0, _G, gbody, 0)
    pooled_ref[...] = pooled_in_ref[...] + jnp.concatenate(
        [gmp_ref[...], gap], axis=1)


def _tc_b(msum, y, dis, batch_row, batchT, bcr, gr, btr, counts, pooled_in):
    h, gap, pooled, _ = pl.pallas_call(
        _tc_b_body,
        out_shape=[
            jax.ShapeDtypeStruct((_N, _H), jnp.float32),
            jax.ShapeDtypeStruct((_G, _H), jnp.float32),
            jax.ShapeDtypeStruct((_G, 2 * _H), jnp.float32),
            jax.ShapeDtypeStruct((_G, _H), jnp.float32),
        ],
    )(msum, y, dis, batch_row, batchT, bcr, gr, btr, counts, pooled_in)
    return h, gap, pooled


def _tc_final_body(p_ref, w1_ref, b1_ref, w2_ref, b2_ref, w3_ref, b3_ref,
                   out_ref):
    o = _leaky(_dot(p_ref[...], w1_ref[...]) + b1_ref[...])
    o = _leaky(_dot(o, w2_ref[...]) + b2_ref[...])
    out_ref[...] = _dot(o, w3_ref[...]) + b3_ref[...]


def _tc_final(pooled, W1, b1r, W2, b2r, W3, b3r):
    return pl.pallas_call(
        _tc_final_body,
        out_shape=jax.ShapeDtypeStruct((_G, 1), jnp.float32),
    )(pooled, W1, b1r, W2, b2r, W3, b3r)


def kernel(x, edge_index, edge_attr, batch, W0, b0, Wc0, bc0, Wc1, bc1,
           Wc2, bc2, g0, bt0, g1, bt1, g2, bt2, W1, b1, W2, b2, W3, b3):
    src4 = edge_index[0].reshape(_NW, _MNB, _MIB, _C)
    dst4 = edge_index[1].reshape(_NW, _MNB, _MIB, _C)
    dst4d = edge_index[1].reshape(_NW, _DNB, _DIB, _DC)
    batch_row = batch.reshape(1, _N)
    batchT = batch.reshape(_N, 1)

    deg_parts = _sc_degree(dst4d)
    h, gap, dis, counts = _tc_init(x, W0, b0.reshape(1, _H), batch_row,
                                   deg_parts)

    pooled = jnp.zeros((_G, 2 * _H), jnp.float32)
    layer_params = [(Wc0, bc0, g0, bt0), (Wc1, bc1, g1, bt1),
                    (Wc2, bc2, g2, bt2)]
    for Wc, bc, g, bt in layer_params:
        y = _tc_a(h, gap, dis, batch_row, Wc[:_H], Wc[_H:])
        msum = _sc_msg(y, src4, dst4)
        h, gap, pooled = _tc_b(msum, y, dis, batch_row, batchT,
                               bc.reshape(1, _H), g.reshape(1, _H),
                               bt.reshape(1, _H), counts, pooled)

    return _tc_final(pooled, W1, b1.reshape(1, 4 * _H),
                     W2, b2.reshape(1, 4 * _H), W3, b3.reshape(1, 1))
